# Initial kernel scaffold; baseline (speedup 1.0000x reference)
import jax
import jax.numpy as jnp
from jax.experimental import pallas as pl


def kernel(time_gaps, pe):
    raise NotImplementedError("write your pallas kernel here")


# speedup vs baseline: 14.9753x; 14.9753x over previous
import functools

import jax
import jax.numpy as jnp
from jax import lax
from jax.experimental import pallas as pl
from jax.experimental.pallas import tpu as pltpu
from jax.experimental.pallas import tpu_sc as plsc

_NC = 2
_NS = 16
_NW = _NC * _NS

_G = 128
_CHUNK = 512
_GPC = _CHUNK // _G


@functools.cache
def _build_gather(B: int, V: int, D: int):
    assert B % (_NW * _CHUNK) == 0
    b_per_w = B // _NW
    iters = b_per_w // _CHUNK
    rows_per_w = b_per_w // _G

    mesh = plsc.VectorSubcoreMesh(core_axis_name="c", subcore_axis_name="s")

    @functools.partial(
        pl.kernel,
        mesh=mesh,
        out_type=jax.ShapeDtypeStruct((B, D), jnp.float32),
        scratch_types=[
            pltpu.VMEM((_GPC, _G), jnp.int32),
            pltpu.VMEM((_CHUNK, D), jnp.float32),
            pltpu.SemaphoreType.DMA,
        ],
    )
    def k(table_hbm, idx_hbm, out_hbm, idx_v, rows_v, sem):
        wid = lax.axis_index("s") * _NC + lax.axis_index("c")
        idx_row0 = wid * rows_per_w
        out0 = wid * b_per_w

        def body(g, carry):
            pltpu.sync_copy(idx_hbm.at[pl.ds(idx_row0 + g * _GPC, _GPC)], idx_v)
            copies = []
            for j in range(_GPC):
                copies.append(
                    pltpu.async_copy(
                        table_hbm.at[idx_v.at[j]],
                        rows_v.at[pl.ds(j * _G, _G)],
                        sem,
                    )
                )
            for c in copies:
                c.wait()
            pltpu.sync_copy(rows_v, out_hbm.at[pl.ds(out0 + g * _CHUNK, _CHUNK)])
            return carry

        lax.fori_loop(0, iters, body, 0)

    return k


def kernel(time_gaps, pe):
    Rr, Cc = time_gaps.shape
    V, D = pe.shape[1], pe.shape[2]
    B = Rr * Cc
    idx = time_gaps.reshape(B // _G, _G).astype(jnp.int32)
    table = pe.reshape(V, D)
    out = _build_gather(B, V, D)(table, idx)
    return out.reshape(Rr, Cc, D)


# speedup vs baseline: 33.6391x; 2.2463x over previous
import functools

import jax
import jax.numpy as jnp
from jax import lax
from jax.experimental import pallas as pl
from jax.experimental.pallas import tpu as pltpu
from jax.experimental.pallas import tpu_sc as plsc

_NC = 2
_NS = 16
_NW = _NC * _NS

_G = 128
_CHUNK = 256
_GPC = _CHUNK // _G
_NBUF = 2


@functools.cache
def _build_gather(B: int, V: int, D: int):
    assert B % (_NW * _CHUNK * _NBUF) == 0
    b_per_w = B // _NW
    iters = b_per_w // (_CHUNK * _NBUF)
    rows_per_w = b_per_w // _G

    mesh = plsc.VectorSubcoreMesh(core_axis_name="c", subcore_axis_name="s")

    @functools.partial(
        pl.kernel,
        mesh=mesh,
        out_type=jax.ShapeDtypeStruct((B, D), jnp.float32),
        scratch_types=[
            pltpu.VMEM_SHARED((V, D), jnp.float32),
            pltpu.VMEM((_NBUF, _GPC, _G), jnp.int32),
            pltpu.VMEM((_NBUF, _CHUNK, D), jnp.float32),
            pltpu.SemaphoreType.DMA,
            pltpu.SemaphoreType.DMA,
            pltpu.SemaphoreType.DMA,
        ],
    )
    def k(table_hbm, idx_hbm, out_hbm, table_sh, idx_v, rows_v, gsem, wsem0,
          wsem1):
        sid = lax.axis_index("s")
        wid = sid * _NC + lax.axis_index("c")
        idx_row0 = wid * rows_per_w
        out0 = wid * b_per_w
        wsems = (wsem0, wsem1)

        for s8 in range(8):
            off = s8 * 128
            size = min(128, V - off)

            @pl.when(sid == s8)
            def _(off=off, size=size):
                pltpu.sync_copy(
                    table_hbm.at[pl.ds(off, size)],
                    table_sh.at[pl.ds(off, size)],
                )

        plsc.subcore_barrier()

        def step(g, b, first):
            buf_rows = rows_v.at[b]
            if not first:
                pltpu.make_async_copy(
                    buf_rows, out_hbm.at[pl.ds(out0, _CHUNK)], wsems[b]
                ).wait()
            pltpu.sync_copy(
                idx_hbm.at[pl.ds(idx_row0 + g * _GPC, _GPC)], idx_v.at[b]
            )
            copies = []
            for j in range(_GPC):
                copies.append(
                    pltpu.async_copy(
                        table_sh.at[idx_v.at[b, j]],
                        buf_rows.at[pl.ds(j * _G, _G)],
                        gsem,
                    )
                )
            for c in copies:
                c.wait()
            pltpu.async_copy(
                buf_rows, out_hbm.at[pl.ds(out0 + g * _CHUNK, _CHUNK)], wsems[b]
            )

        for b in range(_NBUF):
            step(b, b, True)

        def body(go, carry):
            for b in range(_NBUF):
                step(go * _NBUF + b, b, False)
            return carry

        lax.fori_loop(1, iters, body, 0)

        for b in range(_NBUF):
            pltpu.make_async_copy(
                rows_v.at[b], out_hbm.at[pl.ds(out0, _CHUNK)], wsems[b]
            ).wait()

    return k


def kernel(time_gaps, pe):
    Rr, Cc = time_gaps.shape
    V, D = pe.shape[1], pe.shape[2]
    B = Rr * Cc
    idx = time_gaps.reshape(B // _G, _G).astype(jnp.int32)
    table = pe.reshape(V, D)
    out = _build_gather(B, V, D)(table, idx)
    return out.reshape(Rr, Cc, D)


# speedup vs baseline: 41.6087x; 1.2369x over previous
import functools

import jax
import jax.numpy as jnp
from jax import lax
from jax.experimental import pallas as pl
from jax.experimental.pallas import tpu as pltpu
from jax.experimental.pallas import tpu_sc as plsc

_NC = 2
_NS = 16
_NW = _NC * _NS

_G = 128
_CHUNK = 256
_GPC = _CHUNK // _G
_NROW = 2
_NIDX = 4
_UNROLL = 4


@functools.cache
def _build_gather(B: int, V: int, D: int):
    assert B % (_NW * _CHUNK * _UNROLL) == 0
    b_per_w = B // _NW
    steps = b_per_w // _CHUNK
    rows_per_w = b_per_w // _G

    mesh = plsc.VectorSubcoreMesh(core_axis_name="c", subcore_axis_name="s")

    @functools.partial(
        pl.kernel,
        mesh=mesh,
        out_type=jax.ShapeDtypeStruct((B, D), jnp.float32),
        scratch_types=[
            pltpu.VMEM_SHARED((V, D), jnp.float32),
            pltpu.VMEM((_NIDX, _GPC, _G), jnp.int32),
            pltpu.VMEM((_NROW, _CHUNK, D), jnp.float32),
            pltpu.SemaphoreType.DMA,
            pltpu.SemaphoreType.DMA,
            pltpu.SemaphoreType.DMA,
            pltpu.SemaphoreType.DMA,
            pltpu.SemaphoreType.DMA,
            pltpu.SemaphoreType.DMA,
            pltpu.SemaphoreType.DMA,
        ],
    )
    def k(table_hbm, idx_hbm, out_hbm, table_sh, idx_v, rows_v, gsem,
          ws0, ws1, is0, is1, is2, is3):
        sid = lax.axis_index("s")
        wid = sid * _NC + lax.axis_index("c")
        idx_row0 = wid * rows_per_w
        out0 = wid * b_per_w
        wsems = (ws0, ws1)
        isems = (is0, is1, is2, is3)

        for s8 in range(8):
            off = s8 * 128
            size = min(128, V - off)

            @pl.when(sid == s8)
            def _(off=off, size=size):
                pltpu.sync_copy(
                    table_hbm.at[pl.ds(off, size)],
                    table_sh.at[pl.ds(off, size)],
                )

        plsc.subcore_barrier()

        def idx_fetch(g, ib):
            pltpu.async_copy(
                idx_hbm.at[pl.ds(idx_row0 + g * _GPC, _GPC)],
                idx_v.at[ib],
                isems[ib],
            )

        def step(g, rb, ib, wait_write, prefetch):
            buf_rows = rows_v.at[rb]
            if wait_write:
                pltpu.make_async_copy(
                    buf_rows, out_hbm.at[pl.ds(out0, _CHUNK)], wsems[rb]
                ).wait()
            pltpu.make_async_copy(
                idx_hbm.at[pl.ds(idx_row0, _GPC)], idx_v.at[ib], isems[ib]
            ).wait()
            copies = []
            for j in range(_GPC):
                copies.append(
                    pltpu.async_copy(
                        table_sh.at[idx_v.at[ib, j]],
                        buf_rows.at[pl.ds(j * _G, _G)],
                        gsem,
                    )
                )
            for c in copies:
                c.wait()
            if prefetch:
                idx_fetch(g + _NIDX, ib)
            pltpu.async_copy(
                buf_rows, out_hbm.at[pl.ds(out0 + g * _CHUNK, _CHUNK)],
                wsems[rb],
            )

        for p in range(_NIDX):
            idx_fetch(p, p)
        for p in range(_UNROLL):
            step(p, p % _NROW, p % _NIDX, p >= _NROW, True)

        def body(go, carry):
            g0 = go * _UNROLL
            for p in range(_UNROLL):
                step(g0 + p, p % _NROW, p % _NIDX, True, True)
            return carry

        lax.fori_loop(1, steps // _UNROLL - 1, body, 0)

        for p in range(_UNROLL):
            step(steps - _UNROLL + p, p % _NROW, p % _NIDX, True, False)

        for rb in range(_NROW):
            pltpu.make_async_copy(
                rows_v.at[rb], out_hbm.at[pl.ds(out0, _CHUNK)], wsems[rb]
            ).wait()

    return k


def kernel(time_gaps, pe):
    Rr, Cc = time_gaps.shape
    V, D = pe.shape[1], pe.shape[2]
    B = Rr * Cc
    idx = time_gaps.reshape(B // _G, _G).astype(jnp.int32)
    table = pe.reshape(V, D)
    out = _build_gather(B, V, D)(table, idx)
    return out.reshape(Rr, Cc, D)
